# baseline (speedup 1.0000x reference)
import jax
import jax.numpy as jnp
from jax.experimental import pallas as pl


def _add_kernel(x1_ref, x2_ref, w_ref, o_ref):
    h = x1_ref.shape[2]
    o_ref[:, :, :h, :] = x1_ref[...] + w_ref[...][None, None, :h, :]
    o_ref[:, :, h:, :] = x2_ref[...] + w_ref[...][None, None, h:, :]


def kernel(x, W):
    batch, n, t, f = x.shape
    xt = jnp.transpose(x, (0, 2, 1, 3))
    nb = 10000
    ts = 2
    h = nb // 2
    out_t = pl.pallas_call(
        _add_kernel,
        grid=(n // nb, batch, t // ts),
        in_specs=[
            pl.BlockSpec((1, ts, h, f), lambda i, b, s: (b, s, 2 * i, 0)),
            pl.BlockSpec((1, ts, h, f), lambda i, b, s: (b, s, 2 * i + 1, 0)),
            pl.BlockSpec((nb, f), lambda i, b, s: (i, 0)),
        ],
        out_specs=pl.BlockSpec((1, ts, nb, f), lambda i, b, s: (b, s, i, 0)),
        out_shape=jax.ShapeDtypeStruct((batch, t, n, f), x.dtype),
    )(xt, xt, W)
    return jnp.transpose(out_t, (0, 2, 1, 3))
